# baseline (speedup 1.0000x reference)
import functools

import jax
import jax.numpy as jnp
from jax import lax
from jax.experimental import pallas as pl
from jax.experimental.pallas import tpu as pltpu
from jax.experimental.pallas import tpu_sc as plsc

_N_IP = 10000
_N_PORT = 10000
_N_CONN = 50000
_E = 160000
_H = 256
_IN_C = 64
_OUT_C = 16
_T = 8

_EB = 128
_BLK = 512

_NIP_P = 10240
_NPORT_P = 10240
_NCONN_P = 51200


def _sc_segsum(h_src, ss, ds, offw, zeros_blk, n_dst, n_dst_pad):
    dw = n_dst_pad // 32
    dump_stride = (n_dst_pad - n_dst) // 32
    mesh = plsc.VectorSubcoreMesh(core_axis_name="c", subcore_axis_name="s")

    @functools.partial(
        pl.kernel,
        mesh=mesh,
        out_type=jax.ShapeDtypeStruct((n_dst_pad, _H), jnp.float32),
        scratch_types=[
            pltpu.VMEM((_EB, _H), jnp.float32),
            pltpu.VMEM((_EB,), jnp.int32),
            pltpu.VMEM((_EB,), jnp.int32),
            pltpu.VMEM((_EB,), jnp.int32),
            pltpu.VMEM((48,), jnp.int32),
            pltpu.SemaphoreType.DMA,
        ],
        compiler_params=pltpu.CompilerParams(needs_layout_passes=False),
    )
    def seg_kernel(h_hbm, ss_hbm, ds_hbm, off_hbm, z_hbm, out_hbm,
                   rows, sidx_v, dsv, dloc_v, offv, sem):
        c = lax.axis_index("c")
        s = lax.axis_index("s")
        w = c * 16 + s
        dump = n_dst + dump_stride * w
        pltpu.sync_copy(off_hbm, offv)
        iota = lax.iota(jnp.int32, 16)
        e0 = jnp.int32(0)
        e1 = jnp.int32(0)
        for p in range(3):
            part = offv[pl.ds(p * 16, 16)]
            e0 = e0 + jnp.sum(jnp.where(iota + p * 16 == w, part, 0))
            e1 = e1 + jnp.sum(jnp.where(iota + p * 16 == w + 1, part, 0))
        pltpu.sync_copy(z_hbm, out_hbm.at[pl.ds(w * dw, dw)])
        b0 = e0 // _EB
        b1 = (e1 + (_EB - 1)) // _EB

        def body(bj, carry):
            e = bj * _EB
            pltpu.sync_copy(ss_hbm.at[pl.ds(e, _EB)], sidx_v)
            pltpu.sync_copy(ds_hbm.at[pl.ds(e, _EB)], dsv)
            for t in range(_EB // 16):
                d16 = dsv[pl.ds(t * 16, 16)]
                posv = iota + (e + t * 16)
                okm = (posv >= e0) & (posv < e1)
                dloc_v[pl.ds(t * 16, 16)] = jnp.where(okm, d16, dump)
            pltpu.async_copy(h_hbm.at[sidx_v], rows, sem).wait()
            pltpu.sync_copy(rows, out_hbm.at[dloc_v], add=True)
            return carry

        lax.fori_loop(b0, b1, body, 0)

    return seg_kernel(h_src, ss, ds, offw, zeros_blk)


def _dot(a, b):
    return jnp.dot(a, b, preferred_element_type=jnp.float32)


def _gru_block(m, hv, wih, whh, bi, bh):
    gi = _dot(m, wih) + bi
    gh = _dot(hv, whh) + bh
    r = jax.nn.sigmoid(gi[:, :_H] + gh[:, :_H])
    z = jax.nn.sigmoid(gi[:, _H:2 * _H] + gh[:, _H:2 * _H])
    n = jnp.tanh(gi[:, 2 * _H:] + r * gh[:, 2 * _H:])
    return (1.0 - z) * n + z * hv


def _full(shape):
    return pl.BlockSpec(shape, lambda i: tuple(0 for _ in shape))


def _tc_update(seg, rec8, mk8, h, WaT, WbT, bm, WihT, WhhT, bih, bhh):
    np_ = h.shape[0]

    def body(seg_ref, r8, m8, h_ref, wa, wb, bmr, wih, whh, bi, bh, out_ref):
        rec = r8[:, :1]
        mk = m8[:, :1]
        hv = h_ref[...]
        m = rec * _dot(seg_ref[...], wa[...]) \
            + mk * (_dot(hv, wb[...]) + bmr[...])
        out_ref[...] = _gru_block(m, hv, wih[...], whh[...], bi[...], bh[...])

    return pl.pallas_call(
        body,
        grid=(np_ // _BLK,),
        in_specs=[
            pl.BlockSpec((_BLK, _H), lambda i: (i, 0)),
            pl.BlockSpec((_BLK, 8), lambda i: (i, 0)),
            pl.BlockSpec((_BLK, 8), lambda i: (i, 0)),
            pl.BlockSpec((_BLK, _H), lambda i: (i, 0)),
            _full((_H, _H)),
            _full((_H, _H)),
            _full((1, _H)),
            _full((_H, 3 * _H)),
            _full((_H, 3 * _H)),
            _full((1, 3 * _H)),
            _full((1, 3 * _H)),
        ],
        out_specs=pl.BlockSpec((_BLK, _H), lambda i: (i, 0)),
        out_shape=jax.ShapeDtypeStruct((np_, _H), jnp.float32),
    )(seg, rec8, mk8, h, WaT, WbT, bm, WihT, WhhT, bih, bhh)


def _tc_update_port(seg_a, rec8_a, mk8_a, seg_b, rec8_b, mk8_b, h,
                    WaT, WbT, bm, WihT, WhhT, bih, bhh):
    np_ = h.shape[0]

    def body(sa_ref, ra8, ma8, sb_ref, rb8, mb8, h_ref,
             wa, wb, bmr, wih, whh, bi, bh, out_ref):
        hv = h_ref[...]
        dst_term = _dot(hv, wb[...]) + bmr[...]
        m1 = ra8[:, :1] * _dot(sa_ref[...], wa[...]) + ma8[:, :1] * dst_term
        h1 = _gru_block(m1, hv, wih[...], whh[...], bi[...], bh[...])
        m2 = rb8[:, :1] * _dot(sb_ref[...], wa[...]) + mb8[:, :1] * dst_term
        out_ref[...] = _gru_block(m2, h1, wih[...], whh[...], bi[...], bh[...])

    return pl.pallas_call(
        body,
        grid=(np_ // _BLK,),
        in_specs=[
            pl.BlockSpec((_BLK, _H), lambda i: (i, 0)),
            pl.BlockSpec((_BLK, 8), lambda i: (i, 0)),
            pl.BlockSpec((_BLK, 8), lambda i: (i, 0)),
            pl.BlockSpec((_BLK, _H), lambda i: (i, 0)),
            pl.BlockSpec((_BLK, 8), lambda i: (i, 0)),
            pl.BlockSpec((_BLK, 8), lambda i: (i, 0)),
            pl.BlockSpec((_BLK, _H), lambda i: (i, 0)),
            _full((_H, _H)),
            _full((_H, _H)),
            _full((1, _H)),
            _full((_H, 3 * _H)),
            _full((_H, 3 * _H)),
            _full((1, 3 * _H)),
            _full((1, 3 * _H)),
        ],
        out_specs=pl.BlockSpec((_BLK, _H), lambda i: (i, 0)),
        out_shape=jax.ShapeDtypeStruct((np_, _H), jnp.float32),
    )(seg_a, rec8_a, mk8_a, seg_b, rec8_b, mk8_b, h,
      WaT, WbT, bm, WihT, WhhT, bih, bhh)


def _tc_readout(h_conn, W1T, b1, W2T, b2, W3T, b3):
    np_ = h_conn.shape[0]

    def body(h_ref, w1, b1r, w2, b2r, w3, b3r, out_ref):
        r = jax.nn.relu(_dot(h_ref[...], w1[...]) + b1r[...])
        r = jax.nn.relu(_dot(r, w2[...]) + b2r[...])
        lg = _dot(r, w3[...]) + b3r[...]
        mx = jnp.max(lg, axis=1, keepdims=True)
        e = jnp.exp(lg - mx)
        out_ref[...] = e / jnp.sum(e, axis=1, keepdims=True)

    return pl.pallas_call(
        body,
        grid=(np_ // _BLK,),
        in_specs=[
            pl.BlockSpec((_BLK, _H), lambda i: (i, 0)),
            _full((_H, _H)),
            _full((1, _H)),
            _full((_H, 64)),
            _full((1, 64)),
            _full((64, _OUT_C)),
            _full((1, _OUT_C)),
        ],
        out_specs=pl.BlockSpec((_BLK, _OUT_C), lambda i: (i, 0)),
        out_shape=jax.ShapeDtypeStruct((np_, _OUT_C), jnp.float32),
    )(h_conn, W1T, b1, W2T, b2, W3T, b3)


def _prep_relation(sidx, didx, n_dst_pad):
    s32 = sidx.astype(jnp.int32)
    d32 = didx.astype(jnp.int32)
    order = jnp.argsort(d32)
    ss = s32[order]
    ds = d32[order]
    dw = n_dst_pad // 32
    offw = jnp.searchsorted(
        ds, jnp.arange(48, dtype=jnp.int32) * dw).astype(jnp.int32)
    cnt_off = jnp.searchsorted(
        ds, jnp.arange(n_dst_pad + 1, dtype=jnp.int32))
    counts = (cnt_off[1:] - cnt_off[:-1]).astype(jnp.float32)
    rec = 1.0 / jnp.maximum(counts, 1.0)
    mk = (counts > 0).astype(jnp.float32)
    rec8 = jnp.broadcast_to(rec[:, None], (n_dst_pad, 8))
    mk8 = jnp.broadcast_to(mk[:, None], (n_dst_pad, 8))
    return ss, ds, offw, rec8, mk8


def _pad_rows(x, n_pad):
    return jnp.concatenate(
        [x, jnp.zeros((n_pad - x.shape[0], x.shape[1]), x.dtype)], axis=0)


def kernel(x_ip, x_port, x_conn,
           src_ip_to_port, dst_ip_to_port,
           src_port_to_conn, dst_port_to_conn,
           src_conn_to_port, dst_conn_to_port,
           src_port_to_ip, dst_port_to_ip,
           W_msg, b_msg,
           Wih_ip, Whh_ip, bih_ip, bhh_ip,
           Wih_conn, Whh_conn, bih_conn, bhh_conn,
           W1, b1, W2, b2, W3, b3):
    h_ip = _pad_rows(x_ip, _NIP_P)
    h_port = _pad_rows(x_port, _NPORT_P)
    h_conn = _pad_rows(
        jnp.concatenate(
            [x_conn, jnp.zeros((_N_CONN, _H - _IN_C), x_conn.dtype)], axis=1),
        _NCONN_P)

    zeros_320 = jnp.zeros((_NPORT_P // 32, _H), jnp.float32)
    zeros_1600 = jnp.zeros((_NCONN_P // 32, _H), jnp.float32)

    r_i2p = _prep_relation(src_ip_to_port, dst_ip_to_port, _NPORT_P)
    r_p2i = _prep_relation(src_port_to_ip, dst_port_to_ip, _NIP_P)
    r_p2c = _prep_relation(src_port_to_conn, dst_port_to_conn, _NCONN_P)
    r_c2p = _prep_relation(src_conn_to_port, dst_conn_to_port, _NPORT_P)

    WaT = W_msg[:, :_H].T
    WbT = W_msg[:, _H:].T
    bm = b_msg[None, :]
    WihT_ip, WhhT_ip = Wih_ip.T, Whh_ip.T
    bih_ip2, bhh_ip2 = bih_ip[None, :], bhh_ip[None, :]
    WihT_c, WhhT_c = Wih_conn.T, Whh_conn.T
    bih_c2, bhh_c2 = bih_conn[None, :], bhh_conn[None, :]

    def round_body(_, hs):
        h_ip, h_port, h_conn = hs
        seg_i2p = _sc_segsum(h_ip, r_i2p[0], r_i2p[1], r_i2p[2],
                             zeros_320, _N_PORT, _NPORT_P)
        seg_p2i = _sc_segsum(h_port, r_p2i[0], r_p2i[1], r_p2i[2],
                             zeros_320, _N_IP, _NIP_P)
        seg_p2c = _sc_segsum(h_port, r_p2c[0], r_p2c[1], r_p2c[2],
                             zeros_1600, _N_CONN, _NCONN_P)
        seg_c2p = _sc_segsum(h_conn, r_c2p[0], r_c2p[1], r_c2p[2],
                             zeros_320, _N_PORT, _NPORT_P)

        new_h_ip = _tc_update(seg_p2i, r_p2i[3], r_p2i[4], h_ip,
                              WaT, WbT, bm, WihT_ip, WhhT_ip,
                              bih_ip2, bhh_ip2)
        new_h_conn = _tc_update(seg_p2c, r_p2c[3], r_p2c[4], h_conn,
                                WaT, WbT, bm, WihT_c, WhhT_c,
                                bih_c2, bhh_c2)
        new_h_port = _tc_update_port(seg_c2p, r_c2p[3], r_c2p[4],
                                     seg_i2p, r_i2p[3], r_i2p[4], h_port,
                                     WaT, WbT, bm, WihT_ip, WhhT_ip,
                                     bih_ip2, bhh_ip2)
        return new_h_ip, new_h_port, new_h_conn

    h_ip, h_port, h_conn = lax.fori_loop(
        0, _T, round_body, (h_ip, h_port, h_conn))

    out = _tc_readout(h_conn, W1.T, b1[None, :], W2.T, b2[None, :],
                      W3.T, b3[None, :])
    return out[:_N_CONN]


# speedup vs baseline: 1.0555x; 1.0555x over previous
import functools

import jax
import jax.numpy as jnp
from jax import lax
from jax.experimental import pallas as pl
from jax.experimental.pallas import tpu as pltpu
from jax.experimental.pallas import tpu_sc as plsc

_N_IP = 10000
_N_PORT = 10000
_N_CONN = 50000
_E = 160000
_H = 256
_IN_C = 64
_OUT_C = 16
_T = 8

_EB = 128
_NBUF = 3
_BLK = 512

_NIP_P = 10240
_NPORT_P = 10240
_NCONN_P = 51200


def _sc_segsum(h_src, ss, ds, offw, zeros_blk, n_dst, n_dst_pad):
    dw = n_dst_pad // 32
    dump_stride = (n_dst_pad - n_dst) // 32
    mesh = plsc.VectorSubcoreMesh(core_axis_name="c", subcore_axis_name="s")

    @functools.partial(
        pl.kernel,
        mesh=mesh,
        out_type=jax.ShapeDtypeStruct((n_dst_pad, _H), jnp.float32),
        scratch_types=[
            [pltpu.VMEM((_EB, _H), jnp.float32) for _ in range(_NBUF)],
            [pltpu.VMEM((_EB,), jnp.int32) for _ in range(_NBUF)],
            [pltpu.VMEM((_EB,), jnp.int32) for _ in range(_NBUF)],
            pltpu.VMEM((_EB,), jnp.int32),
            pltpu.VMEM((48,), jnp.int32),
            [pltpu.SemaphoreType.DMA for _ in range(_NBUF)],
            [pltpu.SemaphoreType.DMA for _ in range(_NBUF)],
        ],
        compiler_params=pltpu.CompilerParams(needs_layout_passes=False),
    )
    def seg_kernel(h_hbm, ss_hbm, ds_hbm, off_hbm, z_hbm, out_hbm,
                   rows, sidx, dloc, dsv, offv, gsem, ssem):
        c = lax.axis_index("c")
        s = lax.axis_index("s")
        w = c * 16 + s
        dump = n_dst + dump_stride * w
        pltpu.sync_copy(off_hbm, offv)
        iota = lax.iota(jnp.int32, 16)
        e0 = jnp.int32(0)
        e1 = jnp.int32(0)
        for p in range(3):
            part = offv[pl.ds(p * 16, 16)]
            e0 = e0 + jnp.sum(jnp.where(iota + p * 16 == w, part, 0))
            e1 = e1 + jnp.sum(jnp.where(iota + p * 16 == w + 1, part, 0))
        pltpu.sync_copy(z_hbm, out_hbm.at[pl.ds(w * dw, dw)])
        b0 = e0 // _EB
        b1 = (e1 + (_EB - 1)) // _EB
        nb = b1 - b0

        def body(g, carry):
            jg = b0 + g * _NBUF
            for u in range(_NBUF):
                j = jg + u

                @pl.when(j < b1)
                def _(j=j, u=u):
                    @pl.when(g > 0)
                    def _():
                        pltpu.make_async_copy(
                            rows[u], out_hbm.at[dloc[u]], ssem[u]).wait()
                    e = j * _EB
                    pltpu.sync_copy(ss_hbm.at[pl.ds(e, _EB)], sidx[u])
                    pltpu.sync_copy(ds_hbm.at[pl.ds(e, _EB)], dsv)
                    for t in range(_EB // 16):
                        d16 = dsv[pl.ds(t * 16, 16)]
                        posv = iota + (e + t * 16)
                        okm = (posv >= e0) & (posv < e1)
                        dloc[u][pl.ds(t * 16, 16)] = jnp.where(okm, d16, dump)
                    pltpu.make_async_copy(
                        h_hbm.at[sidx[u]], rows[u], gsem[u]).start()
            for u in range(_NBUF):
                j = jg + u

                @pl.when(j < b1)
                def _(j=j, u=u):
                    pltpu.make_async_copy(
                        h_hbm.at[sidx[u]], rows[u], gsem[u]).wait()
                    pltpu.make_async_copy(
                        rows[u], out_hbm.at[dloc[u]], ssem[u]).start(add=True)
            return carry

        lax.fori_loop(0, (nb + _NBUF - 1) // _NBUF, body, 0)
        for u in range(_NBUF):
            @pl.when(nb > u)
            def _(u=u):
                pltpu.make_async_copy(
                    rows[u], out_hbm.at[dloc[u]], ssem[u]).wait()

    return seg_kernel(h_src, ss, ds, offw, zeros_blk)


def _dot(a, b):
    return jnp.dot(a, b, preferred_element_type=jnp.float32)


def _gru_block(m, hv, wih, whh, bi, bh):
    gi = _dot(m, wih) + bi
    gh = _dot(hv, whh) + bh
    r = jax.nn.sigmoid(gi[:, :_H] + gh[:, :_H])
    z = jax.nn.sigmoid(gi[:, _H:2 * _H] + gh[:, _H:2 * _H])
    n = jnp.tanh(gi[:, 2 * _H:] + r * gh[:, 2 * _H:])
    return (1.0 - z) * n + z * hv


def _full(shape):
    return pl.BlockSpec(shape, lambda i: tuple(0 for _ in shape))


def _tc_update(seg, rec8, mk8, h, WaT, WbT, bm, WihT, WhhT, bih, bhh):
    np_ = h.shape[0]

    def body(seg_ref, r8, m8, h_ref, wa, wb, bmr, wih, whh, bi, bh, out_ref):
        rec = r8[:, :1]
        mk = m8[:, :1]
        hv = h_ref[...]
        m = rec * _dot(seg_ref[...], wa[...]) \
            + mk * (_dot(hv, wb[...]) + bmr[...])
        out_ref[...] = _gru_block(m, hv, wih[...], whh[...], bi[...], bh[...])

    return pl.pallas_call(
        body,
        grid=(np_ // _BLK,),
        in_specs=[
            pl.BlockSpec((_BLK, _H), lambda i: (i, 0)),
            pl.BlockSpec((_BLK, 8), lambda i: (i, 0)),
            pl.BlockSpec((_BLK, 8), lambda i: (i, 0)),
            pl.BlockSpec((_BLK, _H), lambda i: (i, 0)),
            _full((_H, _H)),
            _full((_H, _H)),
            _full((1, _H)),
            _full((_H, 3 * _H)),
            _full((_H, 3 * _H)),
            _full((1, 3 * _H)),
            _full((1, 3 * _H)),
        ],
        out_specs=pl.BlockSpec((_BLK, _H), lambda i: (i, 0)),
        out_shape=jax.ShapeDtypeStruct((np_, _H), jnp.float32),
    )(seg, rec8, mk8, h, WaT, WbT, bm, WihT, WhhT, bih, bhh)


def _tc_update_port(seg_a, rec8_a, mk8_a, seg_b, rec8_b, mk8_b, h,
                    WaT, WbT, bm, WihT, WhhT, bih, bhh):
    np_ = h.shape[0]

    def body(sa_ref, ra8, ma8, sb_ref, rb8, mb8, h_ref,
             wa, wb, bmr, wih, whh, bi, bh, out_ref):
        hv = h_ref[...]
        dst_term = _dot(hv, wb[...]) + bmr[...]
        m1 = ra8[:, :1] * _dot(sa_ref[...], wa[...]) + ma8[:, :1] * dst_term
        h1 = _gru_block(m1, hv, wih[...], whh[...], bi[...], bh[...])
        m2 = rb8[:, :1] * _dot(sb_ref[...], wa[...]) + mb8[:, :1] * dst_term
        out_ref[...] = _gru_block(m2, h1, wih[...], whh[...], bi[...], bh[...])

    return pl.pallas_call(
        body,
        grid=(np_ // _BLK,),
        in_specs=[
            pl.BlockSpec((_BLK, _H), lambda i: (i, 0)),
            pl.BlockSpec((_BLK, 8), lambda i: (i, 0)),
            pl.BlockSpec((_BLK, 8), lambda i: (i, 0)),
            pl.BlockSpec((_BLK, _H), lambda i: (i, 0)),
            pl.BlockSpec((_BLK, 8), lambda i: (i, 0)),
            pl.BlockSpec((_BLK, 8), lambda i: (i, 0)),
            pl.BlockSpec((_BLK, _H), lambda i: (i, 0)),
            _full((_H, _H)),
            _full((_H, _H)),
            _full((1, _H)),
            _full((_H, 3 * _H)),
            _full((_H, 3 * _H)),
            _full((1, 3 * _H)),
            _full((1, 3 * _H)),
        ],
        out_specs=pl.BlockSpec((_BLK, _H), lambda i: (i, 0)),
        out_shape=jax.ShapeDtypeStruct((np_, _H), jnp.float32),
    )(seg_a, rec8_a, mk8_a, seg_b, rec8_b, mk8_b, h,
      WaT, WbT, bm, WihT, WhhT, bih, bhh)


def _tc_readout(h_conn, W1T, b1, W2T, b2, W3T, b3):
    np_ = h_conn.shape[0]

    def body(h_ref, w1, b1r, w2, b2r, w3, b3r, out_ref):
        r = jax.nn.relu(_dot(h_ref[...], w1[...]) + b1r[...])
        r = jax.nn.relu(_dot(r, w2[...]) + b2r[...])
        lg = _dot(r, w3[...]) + b3r[...]
        mx = jnp.max(lg, axis=1, keepdims=True)
        e = jnp.exp(lg - mx)
        out_ref[...] = e / jnp.sum(e, axis=1, keepdims=True)

    return pl.pallas_call(
        body,
        grid=(np_ // _BLK,),
        in_specs=[
            pl.BlockSpec((_BLK, _H), lambda i: (i, 0)),
            _full((_H, _H)),
            _full((1, _H)),
            _full((_H, 64)),
            _full((1, 64)),
            _full((64, _OUT_C)),
            _full((1, _OUT_C)),
        ],
        out_specs=pl.BlockSpec((_BLK, _OUT_C), lambda i: (i, 0)),
        out_shape=jax.ShapeDtypeStruct((np_, _OUT_C), jnp.float32),
    )(h_conn, W1T, b1, W2T, b2, W3T, b3)


def _prep_relation(sidx, didx, n_dst_pad):
    s32 = sidx.astype(jnp.int32)
    d32 = didx.astype(jnp.int32)
    order = jnp.argsort(d32)
    ss = s32[order]
    ds = d32[order]
    dw = n_dst_pad // 32
    offw = jnp.searchsorted(
        ds, jnp.arange(48, dtype=jnp.int32) * dw).astype(jnp.int32)
    cnt_off = jnp.searchsorted(
        ds, jnp.arange(n_dst_pad + 1, dtype=jnp.int32))
    counts = (cnt_off[1:] - cnt_off[:-1]).astype(jnp.float32)
    rec = 1.0 / jnp.maximum(counts, 1.0)
    mk = (counts > 0).astype(jnp.float32)
    rec8 = jnp.broadcast_to(rec[:, None], (n_dst_pad, 8))
    mk8 = jnp.broadcast_to(mk[:, None], (n_dst_pad, 8))
    return ss, ds, offw, rec8, mk8


def _pad_rows(x, n_pad):
    return jnp.concatenate(
        [x, jnp.zeros((n_pad - x.shape[0], x.shape[1]), x.dtype)], axis=0)


def kernel(x_ip, x_port, x_conn,
           src_ip_to_port, dst_ip_to_port,
           src_port_to_conn, dst_port_to_conn,
           src_conn_to_port, dst_conn_to_port,
           src_port_to_ip, dst_port_to_ip,
           W_msg, b_msg,
           Wih_ip, Whh_ip, bih_ip, bhh_ip,
           Wih_conn, Whh_conn, bih_conn, bhh_conn,
           W1, b1, W2, b2, W3, b3):
    h_ip = _pad_rows(x_ip, _NIP_P)
    h_port = _pad_rows(x_port, _NPORT_P)
    h_conn = _pad_rows(
        jnp.concatenate(
            [x_conn, jnp.zeros((_N_CONN, _H - _IN_C), x_conn.dtype)], axis=1),
        _NCONN_P)

    zeros_320 = jnp.zeros((_NPORT_P // 32, _H), jnp.float32)
    zeros_1600 = jnp.zeros((_NCONN_P // 32, _H), jnp.float32)

    r_i2p = _prep_relation(src_ip_to_port, dst_ip_to_port, _NPORT_P)
    r_p2i = _prep_relation(src_port_to_ip, dst_port_to_ip, _NIP_P)
    r_p2c = _prep_relation(src_port_to_conn, dst_port_to_conn, _NCONN_P)
    r_c2p = _prep_relation(src_conn_to_port, dst_conn_to_port, _NPORT_P)

    WaT = W_msg[:, :_H].T
    WbT = W_msg[:, _H:].T
    bm = b_msg[None, :]
    WihT_ip, WhhT_ip = Wih_ip.T, Whh_ip.T
    bih_ip2, bhh_ip2 = bih_ip[None, :], bhh_ip[None, :]
    WihT_c, WhhT_c = Wih_conn.T, Whh_conn.T
    bih_c2, bhh_c2 = bih_conn[None, :], bhh_conn[None, :]

    def round_body(_, hs):
        h_ip, h_port, h_conn = hs
        seg_i2p = _sc_segsum(h_ip, r_i2p[0], r_i2p[1], r_i2p[2],
                             zeros_320, _N_PORT, _NPORT_P)
        seg_p2i = _sc_segsum(h_port, r_p2i[0], r_p2i[1], r_p2i[2],
                             zeros_320, _N_IP, _NIP_P)
        seg_p2c = _sc_segsum(h_port, r_p2c[0], r_p2c[1], r_p2c[2],
                             zeros_1600, _N_CONN, _NCONN_P)
        seg_c2p = _sc_segsum(h_conn, r_c2p[0], r_c2p[1], r_c2p[2],
                             zeros_320, _N_PORT, _NPORT_P)

        new_h_ip = _tc_update(seg_p2i, r_p2i[3], r_p2i[4], h_ip,
                              WaT, WbT, bm, WihT_ip, WhhT_ip,
                              bih_ip2, bhh_ip2)
        new_h_conn = _tc_update(seg_p2c, r_p2c[3], r_p2c[4], h_conn,
                                WaT, WbT, bm, WihT_c, WhhT_c,
                                bih_c2, bhh_c2)
        new_h_port = _tc_update_port(seg_c2p, r_c2p[3], r_c2p[4],
                                     seg_i2p, r_i2p[3], r_i2p[4], h_port,
                                     WaT, WbT, bm, WihT_ip, WhhT_ip,
                                     bih_ip2, bhh_ip2)
        return new_h_ip, new_h_port, new_h_conn

    h_ip, h_port, h_conn = lax.fori_loop(
        0, _T, round_body, (h_ip, h_port, h_conn))

    out = _tc_readout(h_conn, W1.T, b1[None, :], W2.T, b2[None, :],
                      W3.T, b3[None, :])
    return out[:_N_CONN]
